# Initial kernel scaffold; baseline (speedup 1.0000x reference)
import jax
import jax.numpy as jnp
from jax.experimental import pallas as pl


def kernel(histogram, x_lims, y_lims, shift):
    raise NotImplementedError("write your pallas kernel here")


# speedup vs baseline: 2.6622x; 2.6622x over previous
import functools

import jax
import jax.numpy as jnp
from jax.experimental import pallas as pl
from jax.experimental.pallas import tpu as pltpu

F = 8


def _body(params_ref, hist_ref, w_ref, supp_ref, acc_ref, tot_ref, *, T, PR, PH, PW):
    b = pl.program_id(0)
    t = pl.program_id(1)
    x = hist_ref[0]
    s1 = jnp.sum(x.reshape(PR, F, PW * F), axis=1)
    pooled = jnp.sum(s1.reshape(PR, PW, F), axis=2)
    acc_ref[pl.ds(t * PR, PR), :] = pooled
    tile_sum = jnp.sum(pooled)

    @pl.when(t == 0)
    def _init():
        tot_ref[0] = tile_sum

    @pl.when(t != 0)
    def _accum():
        tot_ref[0] = tot_ref[0] + tile_sum

    @pl.when(t == T - 1)
    def _finish():
        total = jnp.maximum(tot_ref[0], 1e-12)
        w_ref[0] = acc_ref[:] * (1.0 / total)
        px0 = params_ref[b, 0]
        sx = params_ref[b, 1]
        py0 = params_ref[b, 2]
        sy = params_ref[b, 3]
        c = jax.lax.broadcasted_iota(jnp.int32, (PH, 2 * PW), 1)
        r = jax.lax.broadcasted_iota(jnp.int32, (PH, 2 * PW), 0)
        xv = px0 + (jnp.floor_divide(c, 2).astype(jnp.float32) + 0.5) * sx
        yv = py0 + (r.astype(jnp.float32) + 0.5) * sy
        supp_ref[0] = jnp.where(c % 2 == 0, xv, yv)


def kernel(histogram, x_lims, y_lims, shift):
    B, H, W = histogram.shape
    PH, PW = H // F, W // F
    RT = min(512, H)
    T = H // RT
    PR = RT // F

    sx = (x_lims[:, 1] - x_lims[:, 0]) / PW
    sy = (y_lims[:, 1] - y_lims[:, 0]) / PH
    px0 = x_lims[:, 0] + shift[0, 0, 0]
    py0 = y_lims[:, 0] + shift[0, 0, 1]
    params = jnp.stack([px0, sx, py0, sy], axis=-1)

    w, supp2d = pl.pallas_call(
        functools.partial(_body, T=T, PR=PR, PH=PH, PW=PW),
        grid=(B, T),
        in_specs=[
            pl.BlockSpec((B, 4), lambda b, t: (0, 0), memory_space=pltpu.SMEM),
            pl.BlockSpec((1, RT, W), lambda b, t: (b, t, 0)),
        ],
        out_specs=[
            pl.BlockSpec((1, PH, PW), lambda b, t: (b, 0, 0)),
            pl.BlockSpec((1, PH, 2 * PW), lambda b, t: (b, 0, 0)),
        ],
        out_shape=[
            jax.ShapeDtypeStruct((B, PH, PW), jnp.float32),
            jax.ShapeDtypeStruct((B, PH, 2 * PW), jnp.float32),
        ],
        scratch_shapes=[
            pltpu.VMEM((PH, PW), jnp.float32),
            pltpu.SMEM((1,), jnp.float32),
        ],
        compiler_params=pltpu.CompilerParams(
            dimension_semantics=("arbitrary", "arbitrary"),
        ),
    )(params, histogram)

    supp = supp2d.reshape(B, PH * PW, 2)
    weights = w.reshape(B, PH * PW)
    return supp, weights


# speedup vs baseline: 10.1810x; 3.8243x over previous
import functools

import jax
import jax.numpy as jnp
from jax.experimental import pallas as pl
from jax.experimental.pallas import tpu as pltpu

F = 8


def _body(params_ref, hist_ref, w_ref, supp_ref, acc_ref, tot_ref, bmat_ref, *, T, PR, PH, PW):
    b = pl.program_id(0)
    t = pl.program_id(1)
    RT = PR * F
    W = PW * F

    @pl.when((b == 0) & (t == 0))
    def _init_b():
        rr = jax.lax.broadcasted_iota(jnp.int32, (W, PW), 0)
        cc = jax.lax.broadcasted_iota(jnp.int32, (W, PW), 1)
        bmat_ref[:] = (jnp.floor_divide(rr, F) == cc).astype(jnp.float32)

    x = hist_ref[0]
    ar = jax.lax.broadcasted_iota(jnp.int32, (PR, RT), 0)
    ac = jax.lax.broadcasted_iota(jnp.int32, (PR, RT), 1)
    amat = (jnp.floor_divide(ac, F) == ar).astype(jnp.float32)
    s1 = jnp.dot(amat, x, preferred_element_type=jnp.float32)
    pooled = jnp.dot(s1, bmat_ref[:], preferred_element_type=jnp.float32)
    acc_ref[pl.ds(t * PR, PR), :] = pooled
    tile_sum = jnp.sum(pooled)

    @pl.when(t == 0)
    def _init():
        tot_ref[0] = tile_sum

    @pl.when(t != 0)
    def _accum():
        tot_ref[0] = tot_ref[0] + tile_sum

    @pl.when(t == T - 1)
    def _finish():
        total = jnp.maximum(tot_ref[0], 1e-12)
        w_ref[0] = acc_ref[:] * (1.0 / total)
        px0 = params_ref[b, 0]
        sx = params_ref[b, 1]
        py0 = params_ref[b, 2]
        sy = params_ref[b, 3]
        c = jax.lax.broadcasted_iota(jnp.int32, (PH, 2 * PW), 1)
        r = jax.lax.broadcasted_iota(jnp.int32, (PH, 2 * PW), 0)
        xv = px0 + (jnp.floor_divide(c, 2).astype(jnp.float32) + 0.5) * sx
        yv = py0 + (r.astype(jnp.float32) + 0.5) * sy
        supp_ref[0] = jnp.where(c % 2 == 0, xv, yv)


def kernel(histogram, x_lims, y_lims, shift):
    B, H, W = histogram.shape
    PH, PW = H // F, W // F
    RT = min(512, H)
    T = H // RT
    PR = RT // F

    sx = (x_lims[:, 1] - x_lims[:, 0]) / PW
    sy = (y_lims[:, 1] - y_lims[:, 0]) / PH
    px0 = x_lims[:, 0] + shift[0, 0, 0]
    py0 = y_lims[:, 0] + shift[0, 0, 1]
    params = jnp.stack([px0, sx, py0, sy], axis=-1)

    w, supp2d = pl.pallas_call(
        functools.partial(_body, T=T, PR=PR, PH=PH, PW=PW),
        grid=(B, T),
        in_specs=[
            pl.BlockSpec((B, 4), lambda b, t: (0, 0), memory_space=pltpu.SMEM),
            pl.BlockSpec((1, RT, W), lambda b, t: (b, t, 0)),
        ],
        out_specs=[
            pl.BlockSpec((1, PH, PW), lambda b, t: (b, 0, 0)),
            pl.BlockSpec((1, PH, 2 * PW), lambda b, t: (b, 0, 0)),
        ],
        out_shape=[
            jax.ShapeDtypeStruct((B, PH, PW), jnp.float32),
            jax.ShapeDtypeStruct((B, PH, 2 * PW), jnp.float32),
        ],
        scratch_shapes=[
            pltpu.VMEM((PH, PW), jnp.float32),
            pltpu.SMEM((1,), jnp.float32),
            pltpu.VMEM((W, PW), jnp.float32),
        ],
        compiler_params=pltpu.CompilerParams(
            dimension_semantics=("arbitrary", "arbitrary"),
        ),
    )(params, histogram)

    supp = supp2d.reshape(B, PH * PW, 2)
    weights = w.reshape(B, PH * PW)
    return supp, weights


# speedup vs baseline: 28.7610x; 2.8250x over previous
import functools

import jax
import jax.numpy as jnp
from jax.experimental import pallas as pl
from jax.experimental.pallas import tpu as pltpu

F = 8


def _body(params_ref, hist_ref, w_ref, xg_ref, yg_ref, acc_ref, tot_ref, bmat_ref, *, T, PR, PH, PW):
    b = pl.program_id(0)
    t = pl.program_id(1)
    RT = PR * F
    W = PW * F

    @pl.when((b == 0) & (t == 0))
    def _init_b():
        rr = jax.lax.broadcasted_iota(jnp.int32, (W, PW), 0)
        cc = jax.lax.broadcasted_iota(jnp.int32, (W, PW), 1)
        bmat_ref[:] = (jnp.floor_divide(rr, F) == cc).astype(jnp.float32)

    x = hist_ref[0]
    ar = jax.lax.broadcasted_iota(jnp.int32, (PR, RT), 0)
    ac = jax.lax.broadcasted_iota(jnp.int32, (PR, RT), 1)
    amat = (jnp.floor_divide(ac, F) == ar).astype(jnp.float32)
    s1 = jnp.dot(amat, x, preferred_element_type=jnp.float32)
    pooled = jnp.dot(s1, bmat_ref[:], preferred_element_type=jnp.float32)
    acc_ref[pl.ds(t * PR, PR), :] = pooled
    tile_sum = jnp.sum(pooled)

    @pl.when(t == 0)
    def _init():
        tot_ref[0] = tile_sum

    @pl.when(t != 0)
    def _accum():
        tot_ref[0] = tot_ref[0] + tile_sum

    @pl.when(t == T - 1)
    def _finish():
        total = jnp.maximum(tot_ref[0], 1e-12)
        w_ref[0] = acc_ref[:] * (1.0 / total)
        px0 = params_ref[b, 0]
        sx = params_ref[b, 1]
        py0 = params_ref[b, 2]
        sy = params_ref[b, 3]
        c = jax.lax.broadcasted_iota(jnp.int32, (PH, PW), 1).astype(jnp.float32)
        r = jax.lax.broadcasted_iota(jnp.int32, (PH, PW), 0).astype(jnp.float32)
        xg_ref[0] = px0 + (c + 0.5) * sx
        yg_ref[0] = py0 + (r + 0.5) * sy


def kernel(histogram, x_lims, y_lims, shift):
    B, H, W = histogram.shape
    PH, PW = H // F, W // F
    RT = min(512, H)
    T = H // RT
    PR = RT // F

    sx = (x_lims[:, 1] - x_lims[:, 0]) / PW
    sy = (y_lims[:, 1] - y_lims[:, 0]) / PH
    px0 = x_lims[:, 0] + shift[0, 0, 0]
    py0 = y_lims[:, 0] + shift[0, 0, 1]
    params = jnp.stack([px0, sx, py0, sy], axis=-1)

    w, xg, yg = pl.pallas_call(
        functools.partial(_body, T=T, PR=PR, PH=PH, PW=PW),
        grid=(B, T),
        in_specs=[
            pl.BlockSpec((B, 4), lambda b, t: (0, 0), memory_space=pltpu.SMEM),
            pl.BlockSpec((1, RT, W), lambda b, t: (b, t, 0)),
        ],
        out_specs=[
            pl.BlockSpec((1, PH, PW), lambda b, t: (b, 0, 0)),
            pl.BlockSpec((1, PH, PW), lambda b, t: (b, 0, 0)),
            pl.BlockSpec((1, PH, PW), lambda b, t: (b, 0, 0)),
        ],
        out_shape=[
            jax.ShapeDtypeStruct((B, PH, PW), jnp.float32),
            jax.ShapeDtypeStruct((B, PH, PW), jnp.float32),
            jax.ShapeDtypeStruct((B, PH, PW), jnp.float32),
        ],
        scratch_shapes=[
            pltpu.VMEM((PH, PW), jnp.float32),
            pltpu.SMEM((1,), jnp.float32),
            pltpu.VMEM((W, PW), jnp.float32),
        ],
        compiler_params=pltpu.CompilerParams(
            dimension_semantics=("arbitrary", "arbitrary"),
        ),
    )(params, histogram)

    supp = jnp.stack([xg.reshape(B, PH * PW), yg.reshape(B, PH * PW)], axis=-1)
    weights = w.reshape(B, PH * PW)
    return supp, weights


# speedup vs baseline: 30.0448x; 1.0446x over previous
import functools

import jax
import jax.numpy as jnp
from jax.experimental import pallas as pl
from jax.experimental.pallas import tpu as pltpu

F = 8


def _body(params_ref, hist_ref, w_ref, xg_ref, yg_ref, acc_ref, tot_ref, bmat_ref, *, T, PR, PH, PW):
    b = pl.program_id(0)
    t = pl.program_id(1)
    RT = PR * F
    W = PW * F

    @pl.when((b == 0) & (t == 0))
    def _init_b():
        rr = jax.lax.broadcasted_iota(jnp.int32, (W, PW), 0)
        cc = jax.lax.broadcasted_iota(jnp.int32, (W, PW), 1)
        bmat_ref[:] = (jnp.floor_divide(rr, F) == cc).astype(jnp.float32)

    x = hist_ref[0]
    ar = jax.lax.broadcasted_iota(jnp.int32, (PR, RT), 0)
    ac = jax.lax.broadcasted_iota(jnp.int32, (PR, RT), 1)
    amat = (jnp.floor_divide(ac, F) == ar).astype(jnp.float32)
    s1 = jnp.dot(amat, x, preferred_element_type=jnp.float32)
    pooled = jnp.dot(s1, bmat_ref[:], preferred_element_type=jnp.float32)
    acc_ref[pl.ds(t * PR, PR), :] = pooled
    tile_sum = jnp.sum(pooled)

    @pl.when(t == 0)
    def _init():
        tot_ref[0] = tile_sum

    @pl.when(t != 0)
    def _accum():
        tot_ref[0] = tot_ref[0] + tile_sum

    @pl.when(t == T - 1)
    def _finish():
        total = jnp.maximum(tot_ref[0], 1e-12)
        w_ref[0] = acc_ref[:] * (1.0 / total)
        px0 = params_ref[b, 0]
        sx = params_ref[b, 1]
        py0 = params_ref[b, 2]
        sy = params_ref[b, 3]
        c = jax.lax.broadcasted_iota(jnp.int32, (PH, PW), 1).astype(jnp.float32)
        r = jax.lax.broadcasted_iota(jnp.int32, (PH, PW), 0).astype(jnp.float32)
        xg_ref[0] = px0 + (c + 0.5) * sx
        yg_ref[0] = py0 + (r + 0.5) * sy


def kernel(histogram, x_lims, y_lims, shift):
    B, H, W = histogram.shape
    PH, PW = H // F, W // F
    RT = min(1024, H)
    T = H // RT
    PR = RT // F

    sx = (x_lims[:, 1] - x_lims[:, 0]) / PW
    sy = (y_lims[:, 1] - y_lims[:, 0]) / PH
    px0 = x_lims[:, 0] + shift[0, 0, 0]
    py0 = y_lims[:, 0] + shift[0, 0, 1]
    params = jnp.stack([px0, sx, py0, sy], axis=-1)

    w, xg, yg = pl.pallas_call(
        functools.partial(_body, T=T, PR=PR, PH=PH, PW=PW),
        grid=(B, T),
        in_specs=[
            pl.BlockSpec((B, 4), lambda b, t: (0, 0), memory_space=pltpu.SMEM),
            pl.BlockSpec((1, RT, W), lambda b, t: (b, t, 0)),
        ],
        out_specs=[
            pl.BlockSpec((1, PH, PW), lambda b, t: (b, 0, 0)),
            pl.BlockSpec((1, PH, PW), lambda b, t: (b, 0, 0)),
            pl.BlockSpec((1, PH, PW), lambda b, t: (b, 0, 0)),
        ],
        out_shape=[
            jax.ShapeDtypeStruct((B, PH, PW), jnp.float32),
            jax.ShapeDtypeStruct((B, PH, PW), jnp.float32),
            jax.ShapeDtypeStruct((B, PH, PW), jnp.float32),
        ],
        scratch_shapes=[
            pltpu.VMEM((PH, PW), jnp.float32),
            pltpu.SMEM((1,), jnp.float32),
            pltpu.VMEM((W, PW), jnp.float32),
        ],
        compiler_params=pltpu.CompilerParams(
            dimension_semantics=("arbitrary", "arbitrary"),
        ),
    )(params, histogram)

    supp = jnp.stack([xg.reshape(B, PH * PW), yg.reshape(B, PH * PW)], axis=-1)
    weights = w.reshape(B, PH * PW)
    return supp, weights


# speedup vs baseline: 30.3610x; 1.0105x over previous
import functools

import jax
import jax.numpy as jnp
from jax.experimental import pallas as pl
from jax.experimental.pallas import tpu as pltpu

F = 8


def _body(params_ref, hist_ref, w_ref, supp_ref, acc_ref, tot_ref, bmat_ref, *, T, PR, PH, PW):
    b = pl.program_id(0)
    t = pl.program_id(1)
    RT = PR * F
    W = PW * F

    @pl.when((b == 0) & (t == 0))
    def _init_b():
        rr = jax.lax.broadcasted_iota(jnp.int32, (W, PW), 0)
        cc = jax.lax.broadcasted_iota(jnp.int32, (W, PW), 1)
        bmat_ref[:] = (jnp.floor_divide(rr, F) == cc).astype(jnp.float32)

    x = hist_ref[0]
    ar = jax.lax.broadcasted_iota(jnp.int32, (PR, RT), 0)
    ac = jax.lax.broadcasted_iota(jnp.int32, (PR, RT), 1)
    amat = (jnp.floor_divide(ac, F) == ar).astype(jnp.float32)
    s1 = jnp.dot(amat, x, preferred_element_type=jnp.float32)
    pooled = jnp.dot(s1, bmat_ref[:], preferred_element_type=jnp.float32)
    acc_ref[pl.ds(t * PR, PR), :] = pooled
    tile_sum = jnp.sum(pooled)

    @pl.when(t == 0)
    def _init():
        tot_ref[0] = tile_sum

    @pl.when(t != 0)
    def _accum():
        tot_ref[0] = tot_ref[0] + tile_sum

    @pl.when(t == T - 1)
    def _finish():
        total = jnp.maximum(tot_ref[0], 1e-12)
        w_ref[0] = acc_ref[:] * (1.0 / total)
        px0 = params_ref[b, 0]
        sx = params_ref[b, 1]
        py0 = params_ref[b, 2]
        sy = params_ref[b, 3]
        NTile = PH * PW // 128
        CH = 128
        tt = jax.lax.broadcasted_iota(jnp.int32, (CH, 2, 128), 0)
        cd = jax.lax.broadcasted_iota(jnp.int32, (CH, 2, 128), 1)
        ln = jax.lax.broadcasted_iota(jnp.int32, (CH, 2, 128), 2)

        def _chunk(i, carry):
            g = (i * CH + tt) * 128 + ln
            xval = px0 + (jnp.remainder(g, PW).astype(jnp.float32) + 0.5) * sx
            yval = py0 + (jnp.floor_divide(g, PW).astype(jnp.float32) + 0.5) * sy
            supp_ref[0, pl.ds(i * CH, CH)] = jnp.where(cd == 0, xval, yval)
            return carry

        jax.lax.fori_loop(0, NTile // CH, _chunk, 0)


def kernel(histogram, x_lims, y_lims, shift):
    B, H, W = histogram.shape
    PH, PW = H // F, W // F
    RT = min(1024, H)
    T = H // RT
    PR = RT // F

    sx = (x_lims[:, 1] - x_lims[:, 0]) / PW
    sy = (y_lims[:, 1] - y_lims[:, 0]) / PH
    px0 = x_lims[:, 0] + shift[0, 0, 0]
    py0 = y_lims[:, 0] + shift[0, 0, 1]
    params = jnp.stack([px0, sx, py0, sy], axis=-1)

    w, supp4 = pl.pallas_call(
        functools.partial(_body, T=T, PR=PR, PH=PH, PW=PW),
        grid=(B, T),
        in_specs=[
            pl.BlockSpec((B, 4), lambda b, t: (0, 0), memory_space=pltpu.SMEM),
            pl.BlockSpec((1, RT, W), lambda b, t: (b, t, 0)),
        ],
        out_specs=[
            pl.BlockSpec((1, PH, PW), lambda b, t: (b, 0, 0)),
            pl.BlockSpec((1, PH * PW // 128, 2, 128), lambda b, t: (b, 0, 0, 0)),
        ],
        out_shape=[
            jax.ShapeDtypeStruct((B, PH, PW), jnp.float32),
            jax.ShapeDtypeStruct((B, PH * PW // 128, 2, 128), jnp.float32),
        ],
        scratch_shapes=[
            pltpu.VMEM((PH, PW), jnp.float32),
            pltpu.SMEM((1,), jnp.float32),
            pltpu.VMEM((W, PW), jnp.float32),
        ],
        compiler_params=pltpu.CompilerParams(
            dimension_semantics=("arbitrary", "arbitrary"),
        ),
    )(params, histogram)

    supp = supp4.swapaxes(2, 3).reshape(B, PH * PW, 2)
    weights = w.reshape(B, PH * PW)
    return supp, weights
